# Initial kernel scaffold; baseline (speedup 1.0000x reference)
import jax
import jax.numpy as jnp
from jax.experimental import pallas as pl


def kernel(features, edge_index, W1, b1, W2, b2):
    raise NotImplementedError("write your pallas kernel here")


# speedup vs baseline: 3.5585x; 3.5585x over previous
import functools

import jax
import jax.numpy as jnp
from jax import lax
from jax.experimental import pallas as pl
from jax.experimental.pallas import tpu as pltpu
from jax.experimental.pallas import tpu_sc as plsc

N = 10000
E = 320000
NC = 2
NS = 16
NW = NC * NS
CHUNK = 128
EPT = 10240
NCH = EPT // CHUNK
EPAD = NW * EPT
ROWS_T = 632
N_ACC = NS * ROWS_T
DEG_R = 640
N_DEG = NS * DEG_R

@functools.lru_cache(maxsize=None)
def _mesh():
    return plsc.VectorSubcoreMesh(core_axis_name="c", subcore_axis_name="s")


@functools.lru_cache(maxsize=None)
def _make_deg_kernel():
    @functools.partial(
        pl.kernel,
        out_type=jax.ShapeDtypeStruct((NC, 1, N_DEG), jnp.float32),
        mesh=_mesh(),
        scratch_types=[
            pltpu.VMEM((NCH, CHUNK), jnp.int32),
            pltpu.VMEM((CHUNK,), jnp.float32),
            pltpu.VMEM_SHARED((N_DEG,), jnp.float32),
        ],
    )
    def deg_kernel(dst_hbm, ones_hbm, zeros_hbm, out_hbm, dst_v, ones_v, acc):
        c = lax.axis_index("c")
        s = lax.axis_index("s")
        w = c * NS + s
        pltpu.sync_copy(ones_hbm, ones_v)
        pltpu.sync_copy(dst_hbm.at[w], dst_v)
        pltpu.sync_copy(zeros_hbm, acc.at[pl.ds(s * DEG_R, DEG_R)])
        plsc.subcore_barrier()

        @pl.loop(0, NCH)
        def _(j):
            pltpu.sync_copy(ones_v, acc.at[dst_v.at[j]], add=True)

        plsc.subcore_barrier()

        @pl.when(s == 0)
        def _():
            pltpu.sync_copy(acc, out_hbm.at[c, 0])

    return deg_kernel


@functools.lru_cache(maxsize=None)
def _make_prop_kernel(d):
    @functools.partial(
        pl.kernel,
        out_type=jax.ShapeDtypeStruct((NC, N_ACC, d), jnp.float32),
        mesh=_mesh(),
        scratch_types=[
            pltpu.VMEM((NCH, CHUNK), jnp.int32),
            pltpu.VMEM((NCH, CHUNK), jnp.int32),
            pltpu.VMEM((CHUNK, d), jnp.float32),
            pltpu.VMEM_SHARED((N_ACC, d), jnp.float32),
            pltpu.SemaphoreType.DMA,
        ],
    )
    def prop_kernel(x_hbm, src_hbm, dst_hbm, zeros_hbm, out_hbm,
                    src_v, dst_v, rows_v, acc, sem):
        c = lax.axis_index("c")
        s = lax.axis_index("s")
        w = c * NS + s
        pltpu.sync_copy(src_hbm.at[w], src_v)
        pltpu.sync_copy(dst_hbm.at[w], dst_v)
        pltpu.sync_copy(zeros_hbm, acc.at[pl.ds(s * ROWS_T, ROWS_T)])
        plsc.subcore_barrier()

        @pl.loop(0, NCH)
        def _(j):
            pltpu.async_copy(x_hbm.at[src_v.at[j]], rows_v, sem).wait()
            pltpu.sync_copy(rows_v, acc.at[dst_v.at[j]], add=True)

        plsc.subcore_barrier()
        pltpu.sync_copy(acc.at[pl.ds(s * ROWS_T, ROWS_T)],
                        out_hbm.at[c, pl.ds(s * ROWS_T, ROWS_T)])

    return prop_kernel


def _tc_scale_body(deg_ref, x_ref, xn_ref, norm_ref):
    deg = (deg_ref[0, 0, :N] + deg_ref[1, 0, :N]).reshape(N, 1)
    norm = lax.rsqrt(jnp.maximum(deg, 1.0))
    norm_ref[...] = norm
    xn_ref[...] = x_ref[...] * norm


def _tc_dense_body(p_ref, norm_ref, w1_ref, b1_ref, w2_ref, gn_ref):
    agg = (p_ref[0, :N] + p_ref[1, :N]) * norm_ref[...]
    h = jnp.dot(agg, w1_ref[...], preferred_element_type=jnp.float32)
    h = jnp.maximum(h + b1_ref[...], 0.0)
    g = jnp.dot(h, w2_ref[...], preferred_element_type=jnp.float32)
    gn = g * norm_ref[...]
    gn_ref[...] = jnp.concatenate([gn, jnp.zeros_like(gn)], axis=1)


def _tc_out_body(q_ref, norm_ref, b2_ref, out_ref):
    agg = q_ref[0, :N, :64] + q_ref[1, :N, :64]
    out_ref[...] = agg * norm_ref[...] + b2_ref[...]


def _tc_scale(deg_parts, features):
    return pl.pallas_call(
        _tc_scale_body,
        out_shape=(jax.ShapeDtypeStruct((N, 128), jnp.float32),
                   jax.ShapeDtypeStruct((N, 1), jnp.float32)),
    )(deg_parts, features)


def _tc_dense(p, norm, w1, b1, w2):
    return pl.pallas_call(
        _tc_dense_body,
        out_shape=jax.ShapeDtypeStruct((N, 128), jnp.float32),
    )(p, norm, w1, b1, w2)


def _tc_out(q, norm, b2):
    return pl.pallas_call(
        _tc_out_body,
        out_shape=jax.ShapeDtypeStruct((N, 64), jnp.float32),
    )(q, norm, b2)


def kernel(features, edge_index, W1, b1, W2, b2):
    src = edge_index[0].astype(jnp.int32)
    dst = edge_index[1].astype(jnp.int32)
    pad = EPAD - E
    src_p = jnp.concatenate([src, jnp.zeros((pad,), jnp.int32)])
    dst_p = jnp.concatenate([dst, jnp.full((pad,), N, jnp.int32)])
    src_a = src_p.reshape(NW, NCH, CHUNK)
    dst_a = dst_p.reshape(NW, NCH, CHUNK)

    ones1 = jnp.ones((CHUNK,), jnp.float32)
    zeros1 = jnp.zeros((DEG_R,), jnp.float32)
    zeros128 = jnp.zeros((ROWS_T, 128), jnp.float32)

    deg_parts = _make_deg_kernel()(dst_a, ones1, zeros1)
    xn, norm = _tc_scale(deg_parts, features)
    p1 = _make_prop_kernel(128)(xn, src_a, dst_a, zeros128)
    gn = _tc_dense(p1, norm, W1, b1.reshape(1, 128), W2)
    p2 = _make_prop_kernel(128)(gn, src_a, dst_a, zeros128)
    return _tc_out(p2, norm, b2.reshape(1, 64))


# speedup vs baseline: 4.0315x; 1.1329x over previous
import functools

import jax
import jax.numpy as jnp
from jax import lax
from jax.experimental import pallas as pl
from jax.experimental.pallas import tpu as pltpu
from jax.experimental.pallas import tpu_sc as plsc

N = 10000
E = 320000
NC = 2
NS = 16
NW = NC * NS
CHUNK = 128
EPT = 10240
NCH = EPT // CHUNK
HNCH = NCH // 2
EPAD = NW * EPT
ROWS_T = 632
N_ACC = NS * ROWS_T
DEG_R = 640
N_DEG = NS * DEG_R

@functools.lru_cache(maxsize=None)
def _mesh():
    return plsc.VectorSubcoreMesh(core_axis_name="c", subcore_axis_name="s")


@functools.lru_cache(maxsize=None)
def _make_deg_kernel():
    @functools.partial(
        pl.kernel,
        out_type=jax.ShapeDtypeStruct((NC, 1, N_DEG), jnp.float32),
        mesh=_mesh(),
        scratch_types=[
            pltpu.VMEM((NCH, CHUNK), jnp.int32),
            pltpu.VMEM((CHUNK,), jnp.float32),
            pltpu.VMEM_SHARED((N_DEG,), jnp.float32),
        ],
    )
    def deg_kernel(dst_hbm, ones_hbm, zeros_hbm, out_hbm, dst_v, ones_v, acc):
        c = lax.axis_index("c")
        s = lax.axis_index("s")
        w = c * NS + s
        pltpu.sync_copy(ones_hbm, ones_v)
        pltpu.sync_copy(dst_hbm.at[w], dst_v)
        pltpu.sync_copy(zeros_hbm, acc.at[pl.ds(s * DEG_R, DEG_R)])
        plsc.subcore_barrier()

        @pl.loop(0, NCH)
        def _(j):
            pltpu.sync_copy(ones_v, acc.at[dst_v.at[j]], add=True)

        plsc.subcore_barrier()

        @pl.when(s == 0)
        def _():
            pltpu.sync_copy(acc, out_hbm.at[c, 0])

    return deg_kernel


@functools.lru_cache(maxsize=None)
def _make_prop_kernel(d):
    @functools.partial(
        pl.kernel,
        out_type=jax.ShapeDtypeStruct((NC, N_ACC, d), jnp.float32),
        mesh=_mesh(),
        scratch_types=[
            pltpu.VMEM((HNCH, CHUNK), jnp.int32),
            pltpu.VMEM((HNCH, CHUNK), jnp.int32),
            pltpu.VMEM((CHUNK, d), jnp.float32),
            pltpu.VMEM((CHUNK, d), jnp.float32),
            pltpu.VMEM_SHARED((N_ACC, d), jnp.float32),
            pltpu.SemaphoreType.DMA,
            pltpu.SemaphoreType.DMA,
        ],
    )
    def prop_kernel(x_hbm, src_hbm, dst_hbm, zeros_hbm, out_hbm,
                    src_v, dst_v, rows0, rows1, acc, g0, g1):
        c = lax.axis_index("c")
        s = lax.axis_index("s")
        w = c * NS + s
        pltpu.sync_copy(zeros_hbm, acc.at[pl.ds(s * ROWS_T, ROWS_T)])
        plsc.subcore_barrier()

        for h in range(2):
            pltpu.sync_copy(src_hbm.at[w, pl.ds(h * HNCH, HNCH)], src_v)
            pltpu.sync_copy(dst_hbm.at[w, pl.ds(h * HNCH, HNCH)], dst_v)
            pltpu.async_copy(x_hbm.at[src_v.at[0]], rows0, g0)
            pltpu.async_copy(x_hbm.at[src_v.at[1]], rows1, g1)

            @pl.loop(0, HNCH, step=2)
            def _(j):
                pltpu.make_async_copy(x_hbm.at[src_v.at[j]], rows0, g0).wait()
                pltpu.sync_copy(rows0, acc.at[dst_v.at[j]], add=True)

                @pl.when(j + 2 < HNCH)
                def _():
                    pltpu.async_copy(x_hbm.at[src_v.at[j + 2]], rows0, g0)

                pltpu.make_async_copy(x_hbm.at[src_v.at[j + 1]], rows1, g1).wait()
                pltpu.sync_copy(rows1, acc.at[dst_v.at[j + 1]], add=True)

                @pl.when(j + 3 < HNCH)
                def _():
                    pltpu.async_copy(x_hbm.at[src_v.at[j + 3]], rows1, g1)

        plsc.subcore_barrier()
        pltpu.sync_copy(acc.at[pl.ds(s * ROWS_T, ROWS_T)],
                        out_hbm.at[c, pl.ds(s * ROWS_T, ROWS_T)])

    return prop_kernel


def _tc_scale_body(deg_ref, x_ref, xn_ref, norm_ref):
    deg = (deg_ref[0, 0, :N] + deg_ref[1, 0, :N]).reshape(N, 1)
    norm = lax.rsqrt(jnp.maximum(deg, 1.0))
    norm_ref[...] = norm
    xn_ref[...] = x_ref[...] * norm


def _tc_dense_body(p_ref, norm_ref, w1_ref, b1_ref, w2_ref, gn_ref):
    agg = (p_ref[0, :N] + p_ref[1, :N]) * norm_ref[...]
    h = jnp.dot(agg, w1_ref[...], preferred_element_type=jnp.float32)
    h = jnp.maximum(h + b1_ref[...], 0.0)
    g = jnp.dot(h, w2_ref[...], preferred_element_type=jnp.float32)
    gn = g * norm_ref[...]
    gn_ref[...] = jnp.concatenate([gn, jnp.zeros_like(gn)], axis=1)


def _tc_out_body(q_ref, norm_ref, b2_ref, out_ref):
    agg = q_ref[0, :N, :64] + q_ref[1, :N, :64]
    out_ref[...] = agg * norm_ref[...] + b2_ref[...]


def _tc_scale(deg_parts, features):
    return pl.pallas_call(
        _tc_scale_body,
        out_shape=(jax.ShapeDtypeStruct((N, 128), jnp.float32),
                   jax.ShapeDtypeStruct((N, 1), jnp.float32)),
    )(deg_parts, features)


def _tc_dense(p, norm, w1, b1, w2):
    return pl.pallas_call(
        _tc_dense_body,
        out_shape=jax.ShapeDtypeStruct((N, 128), jnp.float32),
    )(p, norm, w1, b1, w2)


def _tc_out(q, norm, b2):
    return pl.pallas_call(
        _tc_out_body,
        out_shape=jax.ShapeDtypeStruct((N, 64), jnp.float32),
    )(q, norm, b2)


def kernel(features, edge_index, W1, b1, W2, b2):
    src = edge_index[0].astype(jnp.int32)
    dst = edge_index[1].astype(jnp.int32)
    pad = EPAD - E
    src_p = jnp.concatenate([src, jnp.zeros((pad,), jnp.int32)])
    dst_p = jnp.concatenate([dst, jnp.full((pad,), N, jnp.int32)])
    src_a = src_p.reshape(NW, NCH, CHUNK)
    dst_a = dst_p.reshape(NW, NCH, CHUNK)

    ones1 = jnp.ones((CHUNK,), jnp.float32)
    zeros1 = jnp.zeros((DEG_R,), jnp.float32)
    zeros128 = jnp.zeros((ROWS_T, 128), jnp.float32)

    deg_parts = _make_deg_kernel()(dst_a, ones1, zeros1)
    xn, norm = _tc_scale(deg_parts, features)
    p1 = _make_prop_kernel(128)(xn, src_a, dst_a, zeros128)
    gn = _tc_dense(p1, norm, W1, b1.reshape(1, 128), W2)
    p2 = _make_prop_kernel(128)(gn, src_a, dst_a, zeros128)
    return _tc_out(p2, norm, b2.reshape(1, 64))


# speedup vs baseline: 4.1811x; 1.0371x over previous
import functools

import jax
import jax.numpy as jnp
from jax import lax
from jax.experimental import pallas as pl
from jax.experimental.pallas import tpu as pltpu
from jax.experimental.pallas import tpu_sc as plsc

N = 10000
E = 320000
NC = 2
NS = 16
NW = NC * NS
CHUNK = 128
EPT = 10240
NCH = EPT // CHUNK
STG = 40
ST0 = 3
ST1 = 1
NCH0 = STG * ST0
NCH1 = STG * ST1
EPAD = NW * EPT
ROWS_T = 632
N_ACC = NS * ROWS_T
DEG_R = 640
N_DEG = NS * DEG_R

@functools.lru_cache(maxsize=None)
def _mesh():
    return plsc.VectorSubcoreMesh(core_axis_name="c", subcore_axis_name="s")


@functools.lru_cache(maxsize=None)
def _make_deg_kernel():
    @functools.partial(
        pl.kernel,
        out_type=jax.ShapeDtypeStruct((NC, 1, N_DEG), jnp.float32),
        mesh=_mesh(),
        scratch_types=[
            pltpu.VMEM((NCH, CHUNK), jnp.int32),
            pltpu.VMEM((CHUNK,), jnp.float32),
            pltpu.VMEM_SHARED((N_DEG,), jnp.float32),
        ],
    )
    def deg_kernel(dst_hbm, ones_hbm, zeros_hbm, out_hbm, dst_v, ones_v, acc):
        c = lax.axis_index("c")
        s = lax.axis_index("s")
        w = c * NS + s
        pltpu.sync_copy(ones_hbm, ones_v)
        pltpu.sync_copy(dst_hbm.at[w], dst_v)
        pltpu.sync_copy(zeros_hbm, acc.at[pl.ds(s * DEG_R, DEG_R)])
        plsc.subcore_barrier()

        @pl.loop(0, NCH)
        def _(j):
            pltpu.sync_copy(ones_v, acc.at[dst_v.at[j]], add=True)

        plsc.subcore_barrier()

        @pl.when(s == 0)
        def _():
            pltpu.sync_copy(acc, out_hbm.at[c, 0])

    return deg_kernel


@functools.lru_cache(maxsize=None)
def _make_prop_kernel(d):
    @functools.partial(
        pl.kernel,
        out_type=jax.ShapeDtypeStruct((NC, N_ACC, d), jnp.float32),
        mesh=_mesh(),
        scratch_types=[
            pltpu.VMEM((STG, CHUNK), jnp.int32),
            pltpu.VMEM((STG, CHUNK), jnp.int32),
            pltpu.VMEM((CHUNK, d), jnp.float32),
            pltpu.VMEM((CHUNK, d), jnp.float32),
            pltpu.VMEM_SHARED((N_ACC, d), jnp.float32),
            pltpu.SemaphoreType.DMA,
            pltpu.SemaphoreType.DMA,
        ],
    )
    def prop_kernel(x_hbm, src0_hbm, dst0_hbm, src1_hbm, dst1_hbm,
                    zeros_hbm, out_hbm,
                    src_v, dst_v, rows0, rows1, acc, g0, g1):
        c = lax.axis_index("c")
        s = lax.axis_index("s")
        pltpu.sync_copy(zeros_hbm, acc.at[pl.ds(s * ROWS_T, ROWS_T)])
        plsc.subcore_barrier()

        def run(src_hbm, dst_hbm, stages):
            for h in range(stages):
                pltpu.sync_copy(src_hbm.at[s, pl.ds(h * STG, STG)], src_v)
                pltpu.sync_copy(dst_hbm.at[s, pl.ds(h * STG, STG)], dst_v)
                pltpu.async_copy(x_hbm.at[src_v.at[0]], rows0, g0)
                pltpu.async_copy(x_hbm.at[src_v.at[1]], rows1, g1)

                @pl.loop(0, STG, step=2)
                def _(j):
                    pltpu.make_async_copy(x_hbm.at[src_v.at[j]], rows0, g0).wait()
                    pltpu.sync_copy(rows0, acc.at[dst_v.at[j]], add=True)

                    @pl.when(j + 2 < STG)
                    def _():
                        pltpu.async_copy(x_hbm.at[src_v.at[j + 2]], rows0, g0)

                    pltpu.make_async_copy(x_hbm.at[src_v.at[j + 1]], rows1, g1).wait()
                    pltpu.sync_copy(rows1, acc.at[dst_v.at[j + 1]], add=True)

                    @pl.when(j + 3 < STG)
                    def _():
                        pltpu.async_copy(x_hbm.at[src_v.at[j + 3]], rows1, g1)

        @pl.when(c == 0)
        def _():
            run(src0_hbm, dst0_hbm, ST0)

        @pl.when(c == 1)
        def _():
            run(src1_hbm, dst1_hbm, ST1)

        plsc.subcore_barrier()
        pltpu.sync_copy(acc.at[pl.ds(s * ROWS_T, ROWS_T)],
                        out_hbm.at[c, pl.ds(s * ROWS_T, ROWS_T)])

    return prop_kernel


def _tc_scale_body(deg_ref, x_ref, xn_ref, norm_ref):
    deg = (deg_ref[0, 0, :N] + deg_ref[1, 0, :N]).reshape(N, 1)
    norm = lax.rsqrt(jnp.maximum(deg, 1.0))
    norm_ref[...] = norm
    xn_ref[...] = x_ref[...] * norm


def _tc_dense_body(p_ref, norm_ref, w1_ref, b1_ref, w2_ref, gn_ref):
    agg = (p_ref[0, :N] + p_ref[1, :N]) * norm_ref[...]
    h = jnp.dot(agg, w1_ref[...], preferred_element_type=jnp.float32)
    h = jnp.maximum(h + b1_ref[...], 0.0)
    g = jnp.dot(h, w2_ref[...], preferred_element_type=jnp.float32)
    gn = g * norm_ref[...]
    gn_ref[...] = jnp.concatenate([gn, jnp.zeros_like(gn)], axis=1)


def _tc_out_body(q_ref, norm_ref, b2_ref, out_ref):
    agg = q_ref[0, :N, :64] + q_ref[1, :N, :64]
    out_ref[...] = agg * norm_ref[...] + b2_ref[...]


def _tc_scale(deg_parts, features):
    return pl.pallas_call(
        _tc_scale_body,
        out_shape=(jax.ShapeDtypeStruct((N, 128), jnp.float32),
                   jax.ShapeDtypeStruct((N, 1), jnp.float32)),
    )(deg_parts, features)


def _tc_dense(p, norm, w1, b1, w2):
    return pl.pallas_call(
        _tc_dense_body,
        out_shape=jax.ShapeDtypeStruct((N, 128), jnp.float32),
    )(p, norm, w1, b1, w2)


def _tc_out(q, norm, b2):
    return pl.pallas_call(
        _tc_out_body,
        out_shape=jax.ShapeDtypeStruct((N, 64), jnp.float32),
    )(q, norm, b2)


def kernel(features, edge_index, W1, b1, W2, b2):
    src = edge_index[0].astype(jnp.int32)
    dst = edge_index[1].astype(jnp.int32)
    pad = EPAD - E
    src_p = jnp.concatenate([src, jnp.zeros((pad,), jnp.int32)])
    dst_p = jnp.concatenate([dst, jnp.full((pad,), N, jnp.int32)])
    dst_a = dst_p.reshape(NW, NCH, CHUNK)
    e0 = NS * NCH0 * CHUNK
    src0 = src_p[:e0].reshape(NS, NCH0, CHUNK)
    dst0 = dst_p[:e0].reshape(NS, NCH0, CHUNK)
    src1 = src_p[e0:].reshape(NS, NCH1, CHUNK)
    dst1 = dst_p[e0:].reshape(NS, NCH1, CHUNK)

    ones1 = jnp.ones((CHUNK,), jnp.float32)
    zeros1 = jnp.zeros((DEG_R,), jnp.float32)
    zeros128 = jnp.zeros((ROWS_T, 128), jnp.float32)

    deg_parts = _make_deg_kernel()(dst_a, ones1, zeros1)
    xn, norm = _tc_scale(deg_parts, features)
    p1 = _make_prop_kernel(128)(xn, src0, dst0, src1, dst1, zeros128)
    gn = _tc_dense(p1, norm, W1, b1.reshape(1, 128), W2)
    p2 = _make_prop_kernel(128)(gn, src0, dst0, src1, dst1, zeros128)
    return _tc_out(p2, norm, b2.reshape(1, 64))
